# baseline (speedup 1.0000x reference)
import functools

import jax
import jax.numpy as jnp
from jax.experimental import pallas as pl

K = 32
B = 64
N = 32768
D = 64
SEG = 1024
GB = 8
NT = N // SEG
NG = B // GB


def _score_kernel(x_ref, a_ref, bmat_ref, o_ref):
    x = x_ref[...]
    proj = jax.lax.dot_general(x, bmat_ref[...], (((2,), (0,)), ((), ())),
                               preferred_element_type=jnp.float32)
    y = proj * x
    ones = jnp.ones((1, D), jnp.float32)
    pred = jax.lax.dot_general(ones, y, (((1,), (2,)), ((), ())),
                               preferred_element_type=jnp.float32)
    o_ref[...] = a_ref[...] * pred[0]


def _topk_kernel(s_ref, o_ref):
    def step(i, carry):
        total, consumed = carry
        s = s_ref[...]
        m = jnp.max(s, axis=1, keepdims=True)
        eq = (s == m)
        cnt = jnp.sum(eq.astype(jnp.float32), axis=1, keepdims=True)
        take = jnp.clip(jnp.float32(K) - consumed, 0.0, cnt)
        total = total + jnp.where(take > 0.0, m, 0.0) * take
        consumed = consumed + take
        s_ref[...] = jnp.where(eq, -jnp.inf, s)
        return total, consumed

    z = jnp.zeros((B, 1), jnp.float32)
    total, _ = jax.lax.fori_loop(0, K, step, (z, z))
    o_ref[...] = total * (1.0 / K)


@functools.partial(jax.jit, static_argnames=())
def kernel(ins_emb, attn, W):
    d = W.shape[0]
    bmat = jnp.zeros((D, D), jnp.float32).at[:d, d:].set(W.T)
    attn2 = attn.reshape(B, N)

    scores = pl.pallas_call(
        _score_kernel,
        grid=(NG, NT),
        in_specs=[
            pl.BlockSpec((GB, SEG, D), lambda g, t: (g, t, 0)),
            pl.BlockSpec((GB, SEG), lambda g, t: (g, t)),
            pl.BlockSpec((D, D), lambda g, t: (0, 0)),
        ],
        out_specs=pl.BlockSpec((GB, SEG), lambda g, t: (g, t)),
        out_shape=jax.ShapeDtypeStruct((B, N), jnp.float32),
    )(ins_emb, attn2, bmat)

    out = pl.pallas_call(
        _topk_kernel,
        grid=(1,),
        in_specs=[pl.BlockSpec((B, N), lambda i: (0, 0))],
        out_specs=pl.BlockSpec((B, 1), lambda i: (0, 0)),
        out_shape=jax.ShapeDtypeStruct((B, 1), jnp.float32),
    )(scores)
    return out


# speedup vs baseline: 1.5224x; 1.5224x over previous
import functools

import jax
import jax.numpy as jnp
from jax.experimental import pallas as pl

K = 32
B = 64
N = 32768
D = 64
PK = 8
IPB = 16384
S = (B * N) // IPB
RPB = IPB // PK


def _score_kernel(x_ref, a_ref, bmat_ref, o_ref):
    x = x_ref[0]
    proj = jnp.dot(x, bmat_ref[...], preferred_element_type=jnp.float32)
    y = proj * x
    ones = jnp.ones((1, D), jnp.float32)
    pred1 = jax.lax.dot_general(ones, y, (((1,), (1,)), ((), ())),
                                preferred_element_type=jnp.float32)
    pred = jnp.concatenate(
        [pred1[:, p * RPB:(p + 1) * RPB] for p in range(PK)], axis=0)
    o_ref[0] = a_ref[0] * pred


def _topk_kernel(s_ref, o_ref):
    def step(i, carry):
        total, consumed = carry
        s = s_ref[...].reshape(B, 2 * PK, RPB)
        m = jnp.max(jnp.max(s, axis=2), axis=1)
        eq = s == m[:, None, None]
        ce = eq.astype(jnp.float32)
        cnt = jnp.sum(jnp.sum(ce, axis=2), axis=1)
        take = jnp.clip(jnp.float32(K) - consumed, 0.0, cnt)
        total = total + jnp.where(take > 0.0, m, 0.0) * take
        consumed = consumed + take
        s_ref[...] = jnp.where(eq, -jnp.inf, s).reshape(S, PK, RPB)
        return total, consumed

    z = jnp.zeros((B,), jnp.float32)
    total, _ = jax.lax.fori_loop(0, K, step, (z, z))
    o_ref[...] = total.reshape(B, 1) * (1.0 / K)


@functools.partial(jax.jit, static_argnames=())
def kernel(ins_emb, attn, W):
    d = W.shape[0]
    bmat = jnp.zeros((D, D), jnp.float32).at[:d, d:].set(W.T)
    x8 = ins_emb.reshape(S, IPB, D)
    attn3 = attn.reshape(S, PK, RPB)

    scores = pl.pallas_call(
        _score_kernel,
        grid=(S,),
        in_specs=[
            pl.BlockSpec((1, IPB, D), lambda s: (s, 0, 0)),
            pl.BlockSpec((1, PK, RPB), lambda s: (s, 0, 0)),
            pl.BlockSpec((D, D), lambda s: (0, 0)),
        ],
        out_specs=pl.BlockSpec((1, PK, RPB), lambda s: (s, 0, 0)),
        out_shape=jax.ShapeDtypeStruct((S, PK, RPB), jnp.float32),
    )(x8, attn3, bmat)

    out = pl.pallas_call(
        _topk_kernel,
        grid=(1,),
        in_specs=[pl.BlockSpec((S, PK, RPB), lambda i: (0, 0, 0))],
        out_specs=pl.BlockSpec((B, 1), lambda i: (0, 0)),
        out_shape=jax.ShapeDtypeStruct((B, 1), jnp.float32),
    )(scores)
    return out
